# baseline (speedup 1.0000x reference)
import functools

import jax
import jax.numpy as jnp
from jax import lax
from jax.experimental import pallas as pl
from jax.experimental.pallas import tpu as pltpu
from jax.experimental.pallas import tpu_sc as plsc

TOPK = 5
NEG = -1e30
BLK = 1000


def _relu(v):
    return jnp.maximum(v, 0.0)


def _mm(a, b):
    return lax.dot_general(a, b, (((1,), (0,)), ((), ())),
                           preferred_element_type=jnp.float32)


def _frontend_body(x_ref, query_ref, qmask_ref, batch_ref, W_v_ref, W_u_ref,
                   ins_ref, p0_ref):
    x = x_ref[...]
    query = query_ref[...]
    qmask = qmask_ref[...]
    batch = batch_ref[...]
    W_u = W_u_ref[...]
    N, EMB = x.shape
    B, S, _ = query.shape
    K = W_v_ref.shape[0]

    masked_q = query * qmask
    cnt = jnp.maximum(jnp.sum(qmask, axis=1), 1.0)
    qp = jnp.sum(masked_q, axis=1) / cnt

    mask2 = qmask[..., 0]
    i_prev = jnp.zeros((B, EMB), jnp.float32)
    for k in range(K):
        cf = jnp.concatenate([i_prev, qp, qp * i_prev, qp - i_prev], axis=-1)
        q_k = _mm(cf, W_v_ref[k])
        scores = jnp.sum(q_k[:, None, :] * query * W_u[0][None, None, :],
                         axis=-1)
        scores = jnp.where(mask2 == 0, -1e9, scores)
        mx = jnp.max(scores, axis=-1, keepdims=True)
        ex = jnp.exp(scores - mx)
        u = ex / jnp.sum(ex, axis=-1, keepdims=True)
        i_k = jnp.sum(u[..., None] * query * qmask, axis=1)
        ins_ref[:, k, :] = i_k
        i_prev = i_k

    xn = jnp.sqrt(jnp.sum(x * x, axis=1, keepdims=True))
    qpn = jnp.sqrt(jnp.sum(qp * qp, axis=1, keepdims=True))
    dots = lax.dot_general(x, qp, (((1,), (1,)), ((), ())),
                           preferred_element_type=jnp.float32)
    den = jnp.maximum(xn * qpn.reshape(1, B), 1e-8)
    sims = dots / den
    gid = lax.broadcasted_iota(jnp.int32, (1, B), 1)
    batchoh = batch == gid
    gs0 = jnp.where(batchoh, sims, NEG)
    g = gs0
    thr = jnp.full((1, B), NEG, jnp.float32)
    for _ in range(TOPK):
        thr = jnp.max(g, axis=0, keepdims=True)
        g = jnp.where(g == thr, NEG, g)
    sel = jnp.logical_and(batchoh, gs0 >= thr)
    p0_ref[...] = (jnp.sum(sel.astype(jnp.float32), axis=1, keepdims=True)
                   / TOPK)


def _msg_body(h_ref, p_ref, batch_ref, ins_ref, W_x_ref, m_ref):
    h = h_ref[...]
    p = p_ref[...]
    batch = batch_ref[...]
    ins = ins_ref[...]
    B, K, EMB = ins.shape
    nh = _mm(h, W_x_ref[...])
    pnh = p * nh
    gid = lax.broadcasted_iota(jnp.int32, (1, B), 1)
    batchoh = batch == gid
    for k in range(K):
        acc = jnp.zeros(pnh.shape, jnp.float32)
        for gi in range(B):
            iv = ins[gi:gi + 1, k, :]
            acc = acc + jnp.where(batchoh[:, gi:gi + 1],
                                  _relu(iv * pnh), 0.0)
        m_ref[k] = acc


def _layer_body(h_ref, agg_ref, W_h_ref, w_ref, h_new_ref, s_ref):
    h = h_ref[...]
    EMB = h.shape[1]
    K = agg_ref.shape[0]
    z = _mm(h, W_h_ref[0:EMB, :])
    for k in range(K):
        z = z + _mm(agg_ref[k], W_h_ref[(k + 1) * EMB:(k + 2) * EMB, :])
    h_new = _relu(z)
    h_new_ref[...] = h_new
    s_ref[...] = _mm(h_new, w_ref[...])


def _softmax_body(s_ref, p_ref):
    s = s_ref[...]
    mx = jnp.max(s, axis=0, keepdims=True)
    ex = jnp.exp(s - mx)
    p_ref[...] = ex / jnp.sum(ex, axis=0, keepdims=True)


def _make_sc_scatter(NPAD, EMB, E_pad, K):
    NC, NS = 2, 16
    T = 112
    PB = 8
    NBUF = 3
    LA = NBUF - 1
    per_sub = E_pad // NS
    n_pages = per_sub // (T * PB)
    n_bat = n_pages * PB
    assert per_sub % (T * PB) == 0 and NPAD % NS == 0 and K == NC * 2
    rows_sub = NPAD // NS
    ZT = 64
    assert rows_sub % ZT == 0 and rows_sub % 8 == 0 and n_pages >= 3

    mesh = plsc.VectorSubcoreMesh(core_axis_name="c", subcore_axis_name="s",
                                  num_cores=NC, num_subcores=NS)

    @functools.partial(
        pl.kernel, mesh=mesh,
        out_type=jax.ShapeDtypeStruct((K * NPAD, EMB), jnp.float32),
        scratch_types=[
            pltpu.VMEM((2, PB, T), jnp.int32),
            pltpu.VMEM((2, PB, T), jnp.int32),
            pltpu.VMEM((NBUF, T, EMB), jnp.float32),
            pltpu.VMEM_SHARED((NPAD, EMB), jnp.float32),
            [pltpu.SemaphoreType.DMA] * NBUF,
            [pltpu.SemaphoreType.DMA] * NBUF,
            pltpu.SemaphoreType.DMA,
            pltpu.SemaphoreType.DMA,
        ],
    )
    def sc_scatter(m_hbm, src4_hbm, dst_hbm, out_hbm,
                   isrc, idst, rbuf, acc, gsem, ssem, psem, zsem):
        cid = lax.axis_index("c")
        sid = lax.axis_index("s")
        zeros16 = jnp.zeros((16,), jnp.float32)

        def g_issue(t, j):
            pltpu.async_copy(m_hbm.at[isrc.at[(t // PB) % 2, t % PB]],
                             rbuf.at[j], gsem[j])

        def g_wait(j):
            pltpu.make_async_copy(m_hbm.at[isrc.at[0, 0]], rbuf.at[j],
                                  gsem[j]).wait()

        def s_issue(t, j):
            pltpu.async_copy(rbuf.at[j],
                             acc.at[idst.at[(t // PB) % 2, t % PB]],
                             ssem[j], add=True)

        def s_wait(j):
            pltpu.make_async_copy(m_hbm.at[isrc.at[0, 0]], rbuf.at[j],
                                  ssem[j]).wait()

        def pf_issue(c, pg):
            pltpu.async_copy(src4_hbm.at[c, sid, pg], isrc.at[pg % 2], psem)
            pltpu.async_copy(dst_hbm.at[sid, pg], idst.at[pg % 2], psem)

        def pf_wait():
            pltpu.make_async_copy(src4_hbm.at[0, 0, 0], isrc.at[0],
                                  psem).wait()
            pltpu.make_async_copy(src4_hbm.at[0, 0, 0], idst.at[0],
                                  psem).wait()

        base_row = sid * rows_sub
        for i in range(2):
            c = cid * 2 + i
            row_base = cid * (2 * NPAD) + i * NPAD

            def zfill(r, carry):
                for j in range(EMB // 16):
                    rbuf[0, r, pl.ds(j * 16, 16)] = zeros16
                return carry
            lax.fori_loop(0, ZT, zfill, 0)
            for zt in range(rows_sub // ZT):
                pltpu.async_copy(rbuf.at[0, pl.ds(0, ZT)],
                                 acc.at[pl.ds(base_row + zt * ZT, ZT)], zsem)
            for zt in range(rows_sub // ZT):
                pltpu.make_async_copy(
                    rbuf.at[0, pl.ds(0, ZT)],
                    acc.at[pl.ds(base_row, ZT)], zsem).wait()
            plsc.subcore_barrier()

            pltpu.sync_copy(src4_hbm.at[c, sid, 0], isrc.at[0])
            pltpu.sync_copy(dst_hbm.at[sid, 0], idst.at[0])
            pf_issue(c, 1)
            for t in range(LA):
                g_issue(t, t)

            for t in range(n_bat):
                j = t % NBUF
                g_wait(j)
                s_issue(t, j)
                tn = t + LA
                if tn < n_bat:
                    jn = tn % NBUF
                    if t >= 1:
                        s_wait(jn)
                    if t % PB == 0 and t >= PB and (t // PB) + 1 < n_pages:
                        pf_issue(c, (t // PB) + 1)
                    if tn % PB == 0:
                        pf_wait()
                    g_issue(tn, jn)
            for d in range(min(NBUF, n_bat)):
                s_wait((n_bat - 1 - d) % NBUF)
            plsc.subcore_barrier()

            pltpu.sync_copy(acc.at[pl.ds(base_row, rows_sub)],
                            out_hbm.at[pl.ds(row_base + base_row, rows_sub)])
            plsc.subcore_barrier()

    return sc_scatter


def kernel(x, query, query_mask, batch, edge_index, W_v, W_u, W_ih, W_hh,
           b_ih, b_hh, W_q, W_x, W_h, w):
    N, EMB = x.shape
    B, S, _ = query.shape
    K = W_v.shape[0]
    E = edge_index.shape[1]

    NS = 16
    rows_sub = ((N + NS * 32 - 1) // (NS * 32)) * 32
    NPAD = rows_sub * NS
    nblk = N // BLK
    assert N % BLK == 0

    batch2 = batch.reshape(N, 1).astype(jnp.int32)
    T, PB = 112, 8
    per_sub = E // NS
    per_pad = -(-per_sub // (T * PB)) * (T * PB)
    n_pages = per_pad // (T * PB)
    srcP = jnp.pad(edge_index[0].astype(jnp.int32).reshape(NS, per_sub),
                   ((0, 0), (0, per_pad - per_sub)))
    dstP = jnp.pad(edge_index[1].astype(jnp.int32).reshape(NS, per_sub),
                   ((0, 0), (0, per_pad - per_sub)), constant_values=N)
    src4 = (srcP.reshape(1, NS, n_pages, PB, T)
            + (jnp.arange(K, dtype=jnp.int32) * NPAD).reshape(K, 1, 1, 1, 1))
    dst = dstP.reshape(NS, n_pages, PB, T)
    W_u2 = W_u.reshape(1, EMB)
    w2 = w.reshape(EMB, 1)

    ins, p0 = pl.pallas_call(
        _frontend_body,
        out_shape=[jax.ShapeDtypeStruct((B, K, EMB), jnp.float32),
                   jax.ShapeDtypeStruct((N, 1), jnp.float32)],
    )(x, query, query_mask, batch2, W_v, W_u2)

    msg_call = pl.pallas_call(
        _msg_body,
        grid=(nblk,),
        in_specs=[
            pl.BlockSpec((BLK, EMB), lambda i: (i, 0)),
            pl.BlockSpec((BLK, 1), lambda i: (i, 0)),
            pl.BlockSpec((BLK, 1), lambda i: (i, 0)),
            pl.BlockSpec((B, K, EMB), lambda i: (0, 0, 0)),
            pl.BlockSpec((EMB, EMB), lambda i: (0, 0)),
        ],
        out_specs=pl.BlockSpec((K, BLK, EMB), lambda i: (0, i, 0)),
        out_shape=jax.ShapeDtypeStruct((K, NPAD, EMB), jnp.float32),
    )

    layer_call = pl.pallas_call(
        _layer_body,
        grid=(nblk,),
        in_specs=[
            pl.BlockSpec((BLK, EMB), lambda i: (i, 0)),
            pl.BlockSpec((K, BLK, EMB), lambda i: (0, i, 0)),
            pl.BlockSpec((5 * EMB, EMB), lambda i: (0, 0)),
            pl.BlockSpec((EMB, 1), lambda i: (0, 0)),
        ],
        out_specs=[pl.BlockSpec((BLK, EMB), lambda i: (i, 0)),
                   pl.BlockSpec((BLK, 1), lambda i: (i, 0))],
        out_shape=[jax.ShapeDtypeStruct((N, EMB), jnp.float32),
                   jax.ShapeDtypeStruct((N, 1), jnp.float32)],
    )

    softmax_call = pl.pallas_call(
        _softmax_body,
        out_shape=jax.ShapeDtypeStruct((N, 1), jnp.float32),
    )

    sc_scatter = _make_sc_scatter(NPAD, EMB, per_pad * NS, K)

    m0 = msg_call(x, p0, batch2, ins, W_x[0])
    agg0 = sc_scatter(m0.reshape(K * NPAD, EMB), src4, dst)
    h1, s1 = layer_call(x, agg0.reshape(K, NPAD, EMB), W_h[0], w2)
    p1 = softmax_call(s1)

    m1 = msg_call(h1, p1, batch2, ins, W_x[1])
    agg1 = sc_scatter(m1.reshape(K * NPAD, EMB), src4, dst)
    _, s2 = layer_call(h1, agg1.reshape(K, NPAD, EMB), W_h[1], w2)
    p_out = softmax_call(s2)

    return p_out.reshape(N)


# speedup vs baseline: 2.1122x; 2.1122x over previous
import functools

import jax
import jax.numpy as jnp
from jax import lax
from jax.experimental import pallas as pl
from jax.experimental.pallas import tpu as pltpu
from jax.experimental.pallas import tpu_sc as plsc

TOPK = 5
NEG = -1e30
BLK = 1000


def _relu(v):
    return jnp.maximum(v, 0.0)


def _mm(a, b):
    return lax.dot_general(a, b, (((1,), (0,)), ((), ())),
                           preferred_element_type=jnp.float32)


def _frontend_body(x_ref, query_ref, qmask_ref, batch_ref, W_v_ref, W_u_ref,
                   ins_ref, p0_ref):
    x = x_ref[...]
    query = query_ref[...]
    qmask = qmask_ref[...]
    batch = batch_ref[...]
    W_u = W_u_ref[...]
    N, EMB = x.shape
    B, S, _ = query.shape
    K = W_v_ref.shape[0]

    masked_q = query * qmask
    cnt = jnp.maximum(jnp.sum(qmask, axis=1), 1.0)
    qp = jnp.sum(masked_q, axis=1) / cnt

    mask2 = qmask[..., 0]
    i_prev = jnp.zeros((B, EMB), jnp.float32)
    for k in range(K):
        cf = jnp.concatenate([i_prev, qp, qp * i_prev, qp - i_prev], axis=-1)
        q_k = _mm(cf, W_v_ref[k])
        scores = jnp.sum(q_k[:, None, :] * query * W_u[0][None, None, :],
                         axis=-1)
        scores = jnp.where(mask2 == 0, -1e9, scores)
        mx = jnp.max(scores, axis=-1, keepdims=True)
        ex = jnp.exp(scores - mx)
        u = ex / jnp.sum(ex, axis=-1, keepdims=True)
        i_k = jnp.sum(u[..., None] * query * qmask, axis=1)
        ins_ref[:, k, :] = i_k
        i_prev = i_k

    xn = jnp.sqrt(jnp.sum(x * x, axis=1, keepdims=True))
    qpn = jnp.sqrt(jnp.sum(qp * qp, axis=1, keepdims=True))
    dots = lax.dot_general(x, qp, (((1,), (1,)), ((), ())),
                           preferred_element_type=jnp.float32)
    den = jnp.maximum(xn * qpn.reshape(1, B), 1e-8)
    sims = dots / den
    gid = lax.broadcasted_iota(jnp.int32, (1, B), 1)
    batchoh = batch == gid
    gs0 = jnp.where(batchoh, sims, NEG)
    g = gs0
    thr = jnp.full((1, B), NEG, jnp.float32)
    for _ in range(TOPK):
        thr = jnp.max(g, axis=0, keepdims=True)
        g = jnp.where(g == thr, NEG, g)
    sel = jnp.logical_and(batchoh, gs0 >= thr)
    p0_ref[...] = (jnp.sum(sel.astype(jnp.float32), axis=1, keepdims=True)
                   / TOPK)


def _msg_body(h_ref, p_ref, batch_ref, ins_ref, W_x_ref, m_ref):
    h = h_ref[...]
    p = p_ref[...]
    batch = batch_ref[...]
    ins = ins_ref[...]
    B, K, EMB = ins.shape
    nh = _mm(h, W_x_ref[...])
    pnh = p * nh
    gid = lax.broadcasted_iota(jnp.int32, (1, B), 1)
    batchoh = batch == gid
    for k in range(K):
        acc = jnp.zeros(pnh.shape, jnp.float32)
        for gi in range(B):
            iv = ins[gi:gi + 1, k, :]
            acc = acc + jnp.where(batchoh[:, gi:gi + 1],
                                  _relu(iv * pnh), 0.0)
        m_ref[k] = acc


def _layer_body(h_ref, agg_ref, W_h_ref, w_ref, h_new_ref, s_ref):
    h = h_ref[...]
    EMB = h.shape[1]
    K = agg_ref.shape[0]
    z = _mm(h, W_h_ref[0:EMB, :])
    for k in range(K):
        z = z + _mm(agg_ref[k], W_h_ref[(k + 1) * EMB:(k + 2) * EMB, :])
    h_new = _relu(z)
    h_new_ref[...] = h_new
    s_ref[...] = _mm(h_new, w_ref[...])


def _softmax_body(s_ref, p_ref):
    s = s_ref[...]
    mx = jnp.max(s, axis=0, keepdims=True)
    ex = jnp.exp(s - mx)
    p_ref[...] = ex / jnp.sum(ex, axis=0, keepdims=True)


def _make_sc_scatter(NPAD, EMB, E_pad, K):
    NC, NS = 2, 16
    T = 80
    PB = 8
    NBUF = 4
    LA = NBUF - 1
    per_sub = E_pad // NS
    n_pages = per_sub // (T * PB)
    n_bat = n_pages * PB
    assert per_sub % (T * PB) == 0 and NPAD % NS == 0 and K == NC * 2
    rows_sub = NPAD // NS
    assert rows_sub % T == 0 and rows_sub % 8 == 0 and n_pages >= 3

    mesh = plsc.VectorSubcoreMesh(core_axis_name="c", subcore_axis_name="s",
                                  num_cores=NC, num_subcores=NS)

    @functools.partial(
        pl.kernel, mesh=mesh,
        out_type=jax.ShapeDtypeStruct((K * NPAD, EMB), jnp.float32),
        scratch_types=[
            pltpu.VMEM((2, PB, T), jnp.int32),
            pltpu.VMEM((2, PB, T), jnp.int32),
            pltpu.VMEM((NBUF, T, EMB), jnp.float32),
            pltpu.VMEM_SHARED((NPAD, EMB), jnp.float32),
            [pltpu.SemaphoreType.DMA] * NBUF,
            [pltpu.SemaphoreType.DMA] * NBUF,
            pltpu.SemaphoreType.DMA,
            pltpu.SemaphoreType.DMA,
        ],
    )
    def sc_scatter(m_hbm, src4_hbm, dst_hbm, out_hbm,
                   isrc, idst, rbuf, acc, gsem, ssem, psem, zsem):
        cid = lax.axis_index("c")
        sid = lax.axis_index("s")
        zeros16 = jnp.zeros((16,), jnp.float32)

        def g_issue(t, j):
            pltpu.async_copy(m_hbm.at[isrc.at[(t // PB) % 2, t % PB]],
                             rbuf.at[j], gsem[j])

        def g_wait(j):
            pltpu.make_async_copy(m_hbm.at[isrc.at[0, 0]], rbuf.at[j],
                                  gsem[j]).wait()

        def s_issue(t, j):
            pltpu.async_copy(rbuf.at[j],
                             acc.at[idst.at[(t // PB) % 2, t % PB]],
                             ssem[j], add=True)

        def s_wait(j):
            pltpu.make_async_copy(m_hbm.at[isrc.at[0, 0]], rbuf.at[j],
                                  ssem[j]).wait()

        def pf_issue(c, pg):
            pltpu.async_copy(src4_hbm.at[c, sid, pg], isrc.at[pg % 2], psem)
            pltpu.async_copy(dst_hbm.at[sid, pg], idst.at[pg % 2], psem)

        def pf_wait():
            pltpu.make_async_copy(src4_hbm.at[0, 0, 0], isrc.at[0],
                                  psem).wait()
            pltpu.make_async_copy(src4_hbm.at[0, 0, 0], idst.at[0],
                                  psem).wait()

        base_row = sid * rows_sub
        for i in range(2):
            c = cid * 2 + i
            row_base = cid * (2 * NPAD) + i * NPAD

            def zfill(r, carry):
                for j in range(EMB // 16):
                    rbuf[0, r, pl.ds(j * 16, 16)] = zeros16
                return carry
            lax.fori_loop(0, T, zfill, 0)
            for zt in range(rows_sub // T):
                pltpu.async_copy(
                    rbuf.at[0], acc.at[pl.ds(base_row + zt * T, T)], zsem)
            for zt in range(rows_sub // T):
                pltpu.make_async_copy(
                    rbuf.at[0], acc.at[pl.ds(base_row, T)], zsem).wait()
            plsc.subcore_barrier()

            pltpu.sync_copy(src4_hbm.at[c, sid, 0], isrc.at[0])
            pltpu.sync_copy(dst_hbm.at[sid, 0], idst.at[0])
            pf_issue(c, 1)
            for t in range(LA):
                g_issue(t, t)

            for t in range(n_bat):
                j = t % NBUF
                g_wait(j)
                s_issue(t, j)
                tn = t + LA
                if tn < n_bat:
                    jn = tn % NBUF
                    if t >= 1:
                        s_wait(jn)
                    if t % PB == 0 and t >= PB and (t // PB) + 1 < n_pages:
                        pf_issue(c, (t // PB) + 1)
                    if tn % PB == 0:
                        pf_wait()
                    g_issue(tn, jn)
            for d in range(min(NBUF, n_bat)):
                s_wait((n_bat - 1 - d) % NBUF)
            plsc.subcore_barrier()

            pltpu.sync_copy(acc.at[pl.ds(base_row, rows_sub)],
                            out_hbm.at[pl.ds(row_base + base_row, rows_sub)])
            plsc.subcore_barrier()

    return sc_scatter


def kernel(x, query, query_mask, batch, edge_index, W_v, W_u, W_ih, W_hh,
           b_ih, b_hh, W_q, W_x, W_h, w):
    N, EMB = x.shape
    B, S, _ = query.shape
    K = W_v.shape[0]
    E = edge_index.shape[1]

    NS = 16
    rows_sub = ((N + NS * 32 - 1) // (NS * 32)) * 32
    NPAD = rows_sub * NS
    nblk = N // BLK
    assert N % BLK == 0

    batch2 = batch.reshape(N, 1).astype(jnp.int32)
    T, PB = 80, 8
    per_sub = E // NS
    per_pad = -(-per_sub // (T * PB)) * (T * PB)
    n_pages = per_pad // (T * PB)
    srcP = jnp.pad(edge_index[0].astype(jnp.int32).reshape(NS, per_sub),
                   ((0, 0), (0, per_pad - per_sub)))
    dstP = jnp.pad(edge_index[1].astype(jnp.int32).reshape(NS, per_sub),
                   ((0, 0), (0, per_pad - per_sub)), constant_values=N)
    src4 = (srcP.reshape(1, NS, n_pages, PB, T)
            + (jnp.arange(K, dtype=jnp.int32) * NPAD).reshape(K, 1, 1, 1, 1))
    dst = dstP.reshape(NS, n_pages, PB, T)
    W_u2 = W_u.reshape(1, EMB)
    w2 = w.reshape(EMB, 1)

    ins, p0 = pl.pallas_call(
        _frontend_body,
        out_shape=[jax.ShapeDtypeStruct((B, K, EMB), jnp.float32),
                   jax.ShapeDtypeStruct((N, 1), jnp.float32)],
    )(x, query, query_mask, batch2, W_v, W_u2)

    msg_call = pl.pallas_call(
        _msg_body,
        grid=(nblk,),
        in_specs=[
            pl.BlockSpec((BLK, EMB), lambda i: (i, 0)),
            pl.BlockSpec((BLK, 1), lambda i: (i, 0)),
            pl.BlockSpec((BLK, 1), lambda i: (i, 0)),
            pl.BlockSpec((B, K, EMB), lambda i: (0, 0, 0)),
            pl.BlockSpec((EMB, EMB), lambda i: (0, 0)),
        ],
        out_specs=pl.BlockSpec((K, BLK, EMB), lambda i: (0, i, 0)),
        out_shape=jax.ShapeDtypeStruct((K, NPAD, EMB), jnp.float32),
    )

    layer_call = pl.pallas_call(
        _layer_body,
        grid=(nblk,),
        in_specs=[
            pl.BlockSpec((BLK, EMB), lambda i: (i, 0)),
            pl.BlockSpec((K, BLK, EMB), lambda i: (0, i, 0)),
            pl.BlockSpec((5 * EMB, EMB), lambda i: (0, 0)),
            pl.BlockSpec((EMB, 1), lambda i: (0, 0)),
        ],
        out_specs=[pl.BlockSpec((BLK, EMB), lambda i: (i, 0)),
                   pl.BlockSpec((BLK, 1), lambda i: (i, 0))],
        out_shape=[jax.ShapeDtypeStruct((N, EMB), jnp.float32),
                   jax.ShapeDtypeStruct((N, 1), jnp.float32)],
    )

    softmax_call = pl.pallas_call(
        _softmax_body,
        out_shape=jax.ShapeDtypeStruct((N, 1), jnp.float32),
    )

    sc_scatter = _make_sc_scatter(NPAD, EMB, per_pad * NS, K)

    m0 = msg_call(x, p0, batch2, ins, W_x[0])
    agg0 = sc_scatter(m0.reshape(K * NPAD, EMB), src4, dst)
    h1, s1 = layer_call(x, agg0.reshape(K, NPAD, EMB), W_h[0], w2)
    p1 = softmax_call(s1)

    m1 = msg_call(h1, p1, batch2, ins, W_x[1])
    agg1 = sc_scatter(m1.reshape(K * NPAD, EMB), src4, dst)
    _, s2 = layer_call(h1, agg1.reshape(K, NPAD, EMB), W_h[1], w2)
    p_out = softmax_call(s2)

    return p_out.reshape(N)


# speedup vs baseline: 2.1357x; 1.0111x over previous
import functools

import jax
import jax.numpy as jnp
from jax import lax
from jax.experimental import pallas as pl
from jax.experimental.pallas import tpu as pltpu
from jax.experimental.pallas import tpu_sc as plsc

TOPK = 5
NEG = -1e30
BLK = 1000


def _relu(v):
    return jnp.maximum(v, 0.0)


def _mm(a, b):
    return lax.dot_general(a, b, (((1,), (0,)), ((), ())),
                           preferred_element_type=jnp.float32)


def _frontend_body(x_ref, query_ref, qmask_ref, batch_ref, W_v_ref, W_u_ref,
                   ins_ref, p0_ref):
    x = x_ref[...]
    query = query_ref[...]
    qmask = qmask_ref[...]
    batch = batch_ref[...]
    W_u = W_u_ref[...]
    N, EMB = x.shape
    B, S, _ = query.shape
    K = W_v_ref.shape[0]

    masked_q = query * qmask
    cnt = jnp.maximum(jnp.sum(qmask, axis=1), 1.0)
    qp = jnp.sum(masked_q, axis=1) / cnt

    mask2 = qmask[..., 0]
    i_prev = jnp.zeros((B, EMB), jnp.float32)
    for k in range(K):
        cf = jnp.concatenate([i_prev, qp, qp * i_prev, qp - i_prev], axis=-1)
        q_k = _mm(cf, W_v_ref[k])
        scores = jnp.sum(q_k[:, None, :] * query * W_u[0][None, None, :],
                         axis=-1)
        scores = jnp.where(mask2 == 0, -1e9, scores)
        mx = jnp.max(scores, axis=-1, keepdims=True)
        ex = jnp.exp(scores - mx)
        u = ex / jnp.sum(ex, axis=-1, keepdims=True)
        i_k = jnp.sum(u[..., None] * query * qmask, axis=1)
        ins_ref[:, k, :] = i_k
        i_prev = i_k

    xn = jnp.sqrt(jnp.sum(x * x, axis=1, keepdims=True))
    qpn = jnp.sqrt(jnp.sum(qp * qp, axis=1, keepdims=True))
    dots = lax.dot_general(x, qp, (((1,), (1,)), ((), ())),
                           preferred_element_type=jnp.float32)
    den = jnp.maximum(xn * qpn.reshape(1, B), 1e-8)
    sims = dots / den
    gid = lax.broadcasted_iota(jnp.int32, (1, B), 1)
    batchoh = batch == gid
    gs0 = jnp.where(batchoh, sims, NEG)
    g = gs0
    thr = jnp.full((1, B), NEG, jnp.float32)
    for _ in range(TOPK):
        thr = jnp.max(g, axis=0, keepdims=True)
        g = jnp.where(g == thr, NEG, g)
    sel = jnp.logical_and(batchoh, gs0 >= thr)
    p0_ref[...] = (jnp.sum(sel.astype(jnp.float32), axis=1, keepdims=True)
                   / TOPK)


def _msg_body(h_ref, p_ref, batch_ref, ins_ref, W_x_ref, m_ref):
    h = h_ref[...]
    p = p_ref[...]
    batch = batch_ref[...]
    ins = ins_ref[...]
    B, K, EMB = ins.shape
    nh = _mm(h, W_x_ref[...])
    pnh = p * nh
    gid = lax.broadcasted_iota(jnp.int32, (1, B), 1)
    batchoh = batch == gid
    for k in range(K):
        acc = jnp.zeros(pnh.shape, jnp.float32)
        for gi in range(B):
            iv = ins[gi:gi + 1, k, :]
            acc = acc + jnp.where(batchoh[:, gi:gi + 1],
                                  _relu(iv * pnh), 0.0)
        m_ref[k] = acc


def _layer_body(h_ref, agg_ref, W_h_ref, w_ref, h_new_ref, s_ref):
    h = h_ref[...]
    EMB = h.shape[1]
    K = agg_ref.shape[0]
    z = _mm(h, W_h_ref[0:EMB, :])
    for k in range(K):
        z = z + _mm(agg_ref[k], W_h_ref[(k + 1) * EMB:(k + 2) * EMB, :])
    h_new = _relu(z)
    h_new_ref[...] = h_new
    s_ref[...] = _mm(h_new, w_ref[...])


def _softmax_body(s_ref, p_ref):
    s = s_ref[...]
    mx = jnp.max(s, axis=0, keepdims=True)
    ex = jnp.exp(s - mx)
    p_ref[...] = ex / jnp.sum(ex, axis=0, keepdims=True)


def _make_sc_scatter(NPAD, EMB, E_pad, K):
    NC, NS = 2, 16
    T = 40
    PB = 16
    NBUF = 8
    LA = NBUF - 1
    per_sub = E_pad // NS
    n_pages = per_sub // (T * PB)
    n_bat = n_pages * PB
    assert per_sub % (T * PB) == 0 and NPAD % NS == 0 and K == NC * 2
    rows_sub = NPAD // NS
    assert rows_sub % T == 0 and rows_sub % 8 == 0 and n_pages >= 3

    mesh = plsc.VectorSubcoreMesh(core_axis_name="c", subcore_axis_name="s",
                                  num_cores=NC, num_subcores=NS)

    @functools.partial(
        pl.kernel, mesh=mesh,
        out_type=jax.ShapeDtypeStruct((K * NPAD, EMB), jnp.float32),
        scratch_types=[
            pltpu.VMEM((2, PB, T), jnp.int32),
            pltpu.VMEM((2, PB, T), jnp.int32),
            pltpu.VMEM((NBUF, T, EMB), jnp.float32),
            pltpu.VMEM_SHARED((NPAD, EMB), jnp.float32),
            [pltpu.SemaphoreType.DMA] * NBUF,
            [pltpu.SemaphoreType.DMA] * NBUF,
            pltpu.SemaphoreType.DMA,
            pltpu.SemaphoreType.DMA,
        ],
    )
    def sc_scatter(m_hbm, src4_hbm, dst_hbm, out_hbm,
                   isrc, idst, rbuf, acc, gsem, ssem, psem, zsem):
        cid = lax.axis_index("c")
        sid = lax.axis_index("s")
        zeros16 = jnp.zeros((16,), jnp.float32)

        def g_issue(t, j):
            pltpu.async_copy(m_hbm.at[isrc.at[(t // PB) % 2, t % PB]],
                             rbuf.at[j], gsem[j])

        def g_wait(j):
            pltpu.make_async_copy(m_hbm.at[isrc.at[0, 0]], rbuf.at[j],
                                  gsem[j]).wait()

        def s_issue(t, j):
            pltpu.async_copy(rbuf.at[j],
                             acc.at[idst.at[(t // PB) % 2, t % PB]],
                             ssem[j], add=True)

        def s_wait(j):
            pltpu.make_async_copy(m_hbm.at[isrc.at[0, 0]], rbuf.at[j],
                                  ssem[j]).wait()

        def pf_issue(c, pg):
            pltpu.async_copy(src4_hbm.at[c, sid, pg], isrc.at[pg % 2], psem)
            pltpu.async_copy(dst_hbm.at[sid, pg], idst.at[pg % 2], psem)

        def pf_wait():
            pltpu.make_async_copy(src4_hbm.at[0, 0, 0], isrc.at[0],
                                  psem).wait()
            pltpu.make_async_copy(src4_hbm.at[0, 0, 0], idst.at[0],
                                  psem).wait()

        base_row = sid * rows_sub
        for i in range(2):
            c = cid * 2 + i
            row_base = cid * (2 * NPAD) + i * NPAD

            def zfill(r, carry):
                for j in range(EMB // 16):
                    rbuf[0, r, pl.ds(j * 16, 16)] = zeros16
                return carry
            lax.fori_loop(0, T, zfill, 0)
            for zt in range(rows_sub // T):
                pltpu.async_copy(
                    rbuf.at[0], acc.at[pl.ds(base_row + zt * T, T)], zsem)
            for zt in range(rows_sub // T):
                pltpu.make_async_copy(
                    rbuf.at[0], acc.at[pl.ds(base_row, T)], zsem).wait()
            plsc.subcore_barrier()

            pltpu.sync_copy(src4_hbm.at[c, sid, 0], isrc.at[0])
            pltpu.sync_copy(dst_hbm.at[sid, 0], idst.at[0])
            pf_issue(c, 1)
            for t in range(LA):
                g_issue(t, t)

            for t in range(n_bat):
                j = t % NBUF
                g_wait(j)
                s_issue(t, j)
                tn = t + LA
                if tn < n_bat:
                    jn = tn % NBUF
                    if t >= 1:
                        s_wait(jn)
                    if t % PB == 0 and t >= PB and (t // PB) + 1 < n_pages:
                        pf_issue(c, (t // PB) + 1)
                    if tn % PB == 0:
                        pf_wait()
                    g_issue(tn, jn)
            for d in range(min(NBUF, n_bat)):
                s_wait((n_bat - 1 - d) % NBUF)
            plsc.subcore_barrier()

            pltpu.sync_copy(acc.at[pl.ds(base_row, rows_sub)],
                            out_hbm.at[pl.ds(row_base + base_row, rows_sub)])
            plsc.subcore_barrier()

    return sc_scatter


def kernel(x, query, query_mask, batch, edge_index, W_v, W_u, W_ih, W_hh,
           b_ih, b_hh, W_q, W_x, W_h, w):
    N, EMB = x.shape
    B, S, _ = query.shape
    K = W_v.shape[0]
    E = edge_index.shape[1]

    NS = 16
    rows_sub = ((N + NS * 32 - 1) // (NS * 32)) * 32
    NPAD = rows_sub * NS
    nblk = N // BLK
    assert N % BLK == 0

    batch2 = batch.reshape(N, 1).astype(jnp.int32)
    T, PB = 40, 16
    per_sub = E // NS
    per_pad = -(-per_sub // (T * PB)) * (T * PB)
    n_pages = per_pad // (T * PB)
    srcP = jnp.pad(edge_index[0].astype(jnp.int32).reshape(NS, per_sub),
                   ((0, 0), (0, per_pad - per_sub)))
    dstP = jnp.pad(edge_index[1].astype(jnp.int32).reshape(NS, per_sub),
                   ((0, 0), (0, per_pad - per_sub)), constant_values=N)
    src4 = (srcP.reshape(1, NS, n_pages, PB, T)
            + (jnp.arange(K, dtype=jnp.int32) * NPAD).reshape(K, 1, 1, 1, 1))
    dst = dstP.reshape(NS, n_pages, PB, T)
    W_u2 = W_u.reshape(1, EMB)
    w2 = w.reshape(EMB, 1)

    ins, p0 = pl.pallas_call(
        _frontend_body,
        out_shape=[jax.ShapeDtypeStruct((B, K, EMB), jnp.float32),
                   jax.ShapeDtypeStruct((N, 1), jnp.float32)],
    )(x, query, query_mask, batch2, W_v, W_u2)

    msg_call = pl.pallas_call(
        _msg_body,
        grid=(nblk,),
        in_specs=[
            pl.BlockSpec((BLK, EMB), lambda i: (i, 0)),
            pl.BlockSpec((BLK, 1), lambda i: (i, 0)),
            pl.BlockSpec((BLK, 1), lambda i: (i, 0)),
            pl.BlockSpec((B, K, EMB), lambda i: (0, 0, 0)),
            pl.BlockSpec((EMB, EMB), lambda i: (0, 0)),
        ],
        out_specs=pl.BlockSpec((K, BLK, EMB), lambda i: (0, i, 0)),
        out_shape=jax.ShapeDtypeStruct((K, NPAD, EMB), jnp.float32),
    )

    layer_call = pl.pallas_call(
        _layer_body,
        grid=(nblk,),
        in_specs=[
            pl.BlockSpec((BLK, EMB), lambda i: (i, 0)),
            pl.BlockSpec((K, BLK, EMB), lambda i: (0, i, 0)),
            pl.BlockSpec((5 * EMB, EMB), lambda i: (0, 0)),
            pl.BlockSpec((EMB, 1), lambda i: (0, 0)),
        ],
        out_specs=[pl.BlockSpec((BLK, EMB), lambda i: (i, 0)),
                   pl.BlockSpec((BLK, 1), lambda i: (i, 0))],
        out_shape=[jax.ShapeDtypeStruct((N, EMB), jnp.float32),
                   jax.ShapeDtypeStruct((N, 1), jnp.float32)],
    )

    softmax_call = pl.pallas_call(
        _softmax_body,
        out_shape=jax.ShapeDtypeStruct((N, 1), jnp.float32),
    )

    sc_scatter = _make_sc_scatter(NPAD, EMB, per_pad * NS, K)

    m0 = msg_call(x, p0, batch2, ins, W_x[0])
    agg0 = sc_scatter(m0.reshape(K * NPAD, EMB), src4, dst)
    h1, s1 = layer_call(x, agg0.reshape(K, NPAD, EMB), W_h[0], w2)
    p1 = softmax_call(s1)

    m1 = msg_call(h1, p1, batch2, ins, W_x[1])
    agg1 = sc_scatter(m1.reshape(K * NPAD, EMB), src4, dst)
    _, s2 = layer_call(h1, agg1.reshape(K, NPAD, EMB), W_h[1], w2)
    p_out = softmax_call(s2)

    return p_out.reshape(N)
